# baseline (speedup 1.0000x reference)
import functools

import jax
import jax.numpy as jnp
from jax.experimental import pallas as pl
from jax.experimental.pallas import tpu as pltpu
from jax.experimental.pallas import tpu_sc as plsc

_D = 128


def _proj_body(x_ref, w_ref, o0_ref, o1_ref, o2_ref, o3_ref):
    d = x_ref.shape[1]
    prod = jnp.dot(x_ref[...], w_ref[...], preferred_element_type=jnp.float32)
    o0_ref[...] = prod[:, :d]
    o1_ref[...] = prod[:, d:2 * d]
    o2_ref[...] = prod[:, 2 * d:3 * d]
    o3_ref[...] = prod[:, 3 * d:]


def _project_tables(nodes, w_cat):
    n, d = nodes.shape
    bn = 2000
    row = lambda i: (i, 0)
    out_t = jax.ShapeDtypeStruct((n, d), jnp.float32)
    return pl.pallas_call(
        _proj_body,
        grid=(n // bn,),
        in_specs=[
            pl.BlockSpec((bn, d), row),
            pl.BlockSpec((d, w_cat.shape[1]), lambda i: (0, 0)),
        ],
        out_specs=[pl.BlockSpec((bn, d), row)] * 4,
        out_shape=[out_t] * 4,
    )(nodes, w_cat)


def _edge_body(ga_ref, gb_ref, e_ref, w1c_ref, b1_ref, w2_ref, b2_ref,
               g_ref, be_ref, ne_ref, newe_ref):
    e = e_ref[...]
    gsum = ga_ref[...] + gb_ref[...]
    h = gsum + jnp.dot(
        e, w1c_ref[...], preferred_element_type=jnp.float32) + b1_ref[...]
    x1 = jnp.maximum(h, 0.0)
    x2 = jnp.dot(x1, w2_ref[...],
                 preferred_element_type=jnp.float32) + b2_ref[...]
    mu = jnp.mean(x2, axis=-1, keepdims=True)
    xc = x2 - mu
    var = jnp.mean(xc * xc, axis=-1, keepdims=True)
    ne = xc * jax.lax.rsqrt(var + 1e-3) * g_ref[...] + be_ref[...]
    ne_ref[...] = ne
    newe_ref[...] = ne + e


def _edge_mlp(ga, gb, e, w1c, b1, w2, b2, g, be, blk_off=0, n_blk=None):
    num_e, d = e.shape
    be_blk = 2000
    if n_blk is None:
        n_blk = num_e // be_blk
    row = lambda i: (i + blk_off, 0)
    full = lambda i: (0, 0)
    ne, newe = pl.pallas_call(
        _edge_body,
        grid=(n_blk,),
        in_specs=[
            pl.BlockSpec((be_blk, d), row),
            pl.BlockSpec((be_blk, d), row),
            pl.BlockSpec((be_blk, d), row),
            pl.BlockSpec((d, d), full),
            pl.BlockSpec((1, d), full),
            pl.BlockSpec((d, d), full),
            pl.BlockSpec((1, d), full),
            pl.BlockSpec((1, d), full),
            pl.BlockSpec((1, d), full),
        ],
        out_specs=[
            pl.BlockSpec((be_blk, d), lambda i: (i, 0)),
            pl.BlockSpec((be_blk, d), lambda i: (i, 0)),
        ],
        out_shape=[
            jax.ShapeDtypeStruct((n_blk * be_blk, d), jnp.float32),
            jax.ShapeDtypeStruct((n_blk * be_blk, d), jnp.float32),
        ],
    )(ga, gb, e, w1c, b1, w2, b2, g, be)
    return ne, newe


def _node_body(x_ref, a0_ref, a1_ref, wa_ref, wb_ref, wc_ref, b1_ref,
               w2_ref, b2_ref, g_ref, be_ref, o_ref):
    x = x_ref[...]
    agg0 = a0_ref[0] + a0_ref[1]
    agg1 = a1_ref[0] + a1_ref[1]
    h = (jnp.dot(x, wa_ref[...], preferred_element_type=jnp.float32)
         + jnp.dot(agg0, wb_ref[...], preferred_element_type=jnp.float32)
         + jnp.dot(agg1, wc_ref[...], preferred_element_type=jnp.float32)
         + b1_ref[...])
    x1 = jnp.maximum(h, 0.0)
    x2 = jnp.dot(x1, w2_ref[...],
                 preferred_element_type=jnp.float32) + b2_ref[...]
    mu = jnp.mean(x2, axis=-1, keepdims=True)
    xc = x2 - mu
    var = jnp.mean(xc * xc, axis=-1, keepdims=True)
    nn = xc * jax.lax.rsqrt(var + 1e-3) * g_ref[...] + be_ref[...]
    o_ref[...] = nn + x


def _node_mlp(nodes, agg0p, agg1p, wa, wb, wc, b1, w2, b2, g, be):
    n, d = nodes.shape
    bn = 2000
    row = lambda i: (i, 0)
    prow = lambda i: (0, i, 0)
    full = lambda i: (0, 0)
    return pl.pallas_call(
        _node_body,
        grid=(n // bn,),
        in_specs=[
            pl.BlockSpec((bn, d), row),
            pl.BlockSpec((2, bn, d), prow),
            pl.BlockSpec((2, bn, d), prow),
            pl.BlockSpec((d, d), full),
            pl.BlockSpec((d, d), full),
            pl.BlockSpec((d, d), full),
            pl.BlockSpec((1, d), full),
            pl.BlockSpec((d, d), full),
            pl.BlockSpec((1, d), full),
            pl.BlockSpec((1, d), full),
            pl.BlockSpec((1, d), full),
        ],
        out_specs=pl.BlockSpec((bn, d), row),
        out_shape=jax.ShapeDtypeStruct((n, d), jnp.float32),
    )(nodes, agg0p, agg1p, wa, wb, wc, b1, w2, b2, g, be)


def _sc_warmup(x):
    mesh = plsc.ScalarSubcoreMesh(axis_name="core", num_cores=2)

    @functools.partial(
        pl.kernel, out_type=jax.ShapeDtypeStruct(x.shape, x.dtype),
        mesh=mesh,
        scratch_types=[pltpu.SMEM((x.shape[1],), x.dtype),
                       pltpu.SemaphoreType.DMA])
    def k(x_ref, o_ref, tmp, sem):
        idx = jax.lax.axis_index("core")
        pltpu.async_copy(x_ref.at[idx], tmp, sem).wait()
        pltpu.async_copy(tmp, o_ref.at[idx], sem).wait()

    return k(x)


_GATHER_W = 160


def _sc_gather_pair(table_a, table_b, senders, receivers):
    num_e = senders.shape[0]
    d = table_a.shape[1]
    n_win = num_e // _GATHER_W
    idx_s = senders.reshape(n_win, 1, _GATHER_W)
    idx_r = receivers.reshape(n_win, 1, _GATHER_W)
    mesh = plsc.VectorSubcoreMesh(core_axis_name="core",
                                  subcore_axis_name="subcore")
    out_t = jax.ShapeDtypeStruct((num_e, d), table_a.dtype)

    @functools.partial(
        pl.kernel, out_type=[out_t, out_t], mesh=mesh,
        scratch_types=[pltpu.SemaphoreType.DMA, pltpu.SemaphoreType.DMA])
    def k(a_hbm, b_hbm, is_hbm, ir_hbm, oa_hbm, ob_hbm, sem_a, sem_b):
        def body(is_v, ir_v, oa_v, ob_v):
            da = pltpu.async_copy(a_hbm.at[is_v.at[0, 0]], oa_v, sem_a)
            db = pltpu.async_copy(b_hbm.at[ir_v.at[0, 0]], ob_v, sem_b)
            da.wait()
            db.wait()

        pltpu.emit_pipeline(
            body,
            grid=(n_win,),
            in_specs=[
                pl.BlockSpec((1, 1, _GATHER_W), lambda i: (i, 0, 0)),
                pl.BlockSpec((1, 1, _GATHER_W), lambda i: (i, 0, 0)),
            ],
            out_specs=[
                pl.BlockSpec((_GATHER_W, d), lambda i: (i, 0)),
                pl.BlockSpec((_GATHER_W, d), lambda i: (i, 0)),
            ],
            core_axis_name=("core", "subcore"),
            dimension_semantics=(pltpu.PARALLEL,),
        )(is_hbm, ir_hbm, oa_hbm, ob_hbm)

    return k(table_a, table_b, idx_s, idx_r)


_SCAT_W = 128


def _sc_segment_sum(ne, receivers, zeros_nd):
    num_e, d = ne.shape
    n = zeros_nd.shape[0]
    n_chunk = num_e // _SCAT_W
    idx3 = receivers.reshape(n_chunk, 1, _SCAT_W)
    mesh = plsc.VectorSubcoreMesh(core_axis_name="core",
                                  subcore_axis_name="subcore")

    @functools.partial(
        pl.kernel,
        out_type=jax.ShapeDtypeStruct((2, n, d), jnp.float32),
        mesh=mesh,
        scratch_types=[pltpu.VMEM_SHARED((n, d), jnp.float32)],
    )
    def k(ne_hbm, idx_hbm, z_hbm, out_hbm, acc_sh):
        cid = jax.lax.axis_index("core")
        sid = jax.lax.axis_index("subcore")

        @pl.when(sid == 0)
        def _():
            pltpu.sync_copy(z_hbm, acc_sh)

        plsc.subcore_barrier()

        def body(idx_v, rows_v):
            pltpu.sync_copy(rows_v, acc_sh.at[idx_v.at[0, 0]], add=True)

        pltpu.emit_pipeline(
            body,
            grid=(n_chunk,),
            in_specs=[
                pl.BlockSpec((1, 1, _SCAT_W), lambda i: (i, 0, 0)),
                pl.BlockSpec((_SCAT_W, d), lambda i: (i, 0)),
            ],
            out_specs=[],
            core_axis_name=("core", "subcore"),
            dimension_semantics=(pltpu.PARALLEL,),
        )(idx_hbm, ne_hbm)

        plsc.subcore_barrier()

        @pl.when(sid == 0)
        def _():
            pltpu.sync_copy(acc_sh, out_hbm.at[cid])

    return k(ne, idx3, zeros_nd)


def kernel(node_features, edge_feat_0, edge_feat_1, senders_0, receivers_0,
           senders_1, receivers_1, e0_W1, e0_b1, e0_W2, e0_b2, e0_g, e0_be,
           e1_W1, e1_b1, e1_W2, e1_b2, e1_g, e1_be, n_W1, n_b1, n_W2, n_b2,
           n_g, n_be):
    d = _D
    n = node_features.shape[0]

    warm = _sc_warmup(jnp.zeros((2, 16), jnp.int32))
    warm_f = warm[0, 0].astype(jnp.float32)

    w_cat = jnp.concatenate(
        [e0_W1[:d], e0_W1[d:2 * d], e1_W1[:d], e1_W1[d:2 * d]], axis=1) + warm_f
    pa0, pb0, pa1, pb1 = _project_tables(node_features, w_cat)

    r2 = lambda v: v.reshape(1, d)

    ga0, gb0 = _sc_gather_pair(pa0, pb0, senders_0, receivers_0)
    ga1, gb1 = _sc_gather_pair(pa1, pb1, senders_1, receivers_1)

    ne0, new_e0 = _edge_mlp(ga0, gb0, edge_feat_0, e0_W1[2 * d:], r2(e0_b1),
                            e0_W2, r2(e0_b2), r2(e0_g), r2(e0_be))
    ne1, new_e1 = _edge_mlp(ga1, gb1, edge_feat_1, e1_W1[2 * d:], r2(e1_b1),
                            e1_W2, r2(e1_b2), r2(e1_g), r2(e1_be))

    zeros_nd = jnp.zeros((n, d), jnp.float32)
    agg0p = _sc_segment_sum(ne0, receivers_0, zeros_nd)
    agg1p = _sc_segment_sum(ne1, receivers_1, zeros_nd)

    new_nodes = _node_mlp(node_features, agg0p, agg1p, n_W1[:d],
                          n_W1[d:2 * d], n_W1[2 * d:], r2(n_b1), n_W2,
                          r2(n_b2), r2(n_g), r2(n_be))
    return (new_nodes, new_e0, new_e1)


# speedup vs baseline: 1.0098x; 1.0098x over previous
import functools

import jax
import jax.numpy as jnp
from jax.experimental import pallas as pl
from jax.experimental.pallas import tpu as pltpu
from jax.experimental.pallas import tpu_sc as plsc

_D = 128


def _proj_body(x_ref, w_ref, o0_ref, o1_ref, o2_ref, o3_ref):
    d = x_ref.shape[1]
    prod = jnp.dot(x_ref[...], w_ref[...], preferred_element_type=jnp.float32)
    o0_ref[...] = prod[:, :d]
    o1_ref[...] = prod[:, d:2 * d]
    o2_ref[...] = prod[:, 2 * d:3 * d]
    o3_ref[...] = prod[:, 3 * d:]


def _project_tables(nodes, w_cat):
    n, d = nodes.shape
    bn = 2000
    row = lambda i: (i, 0)
    out_t = jax.ShapeDtypeStruct((n, d), jnp.float32)
    return pl.pallas_call(
        _proj_body,
        grid=(n // bn,),
        in_specs=[
            pl.BlockSpec((bn, d), row),
            pl.BlockSpec((d, w_cat.shape[1]), lambda i: (0, 0)),
        ],
        out_specs=[pl.BlockSpec((bn, d), row)] * 4,
        out_shape=[out_t] * 4,
    )(nodes, w_cat)


def _edge_body(ga_ref, gb_ref, e_ref, w1c_ref, b1_ref, w2_ref, b2_ref,
               g_ref, be_ref, ne_ref, newe_ref):
    e = e_ref[...]
    gsum = ga_ref[...] + gb_ref[...]
    h = gsum + jnp.dot(
        e, w1c_ref[...], preferred_element_type=jnp.float32) + b1_ref[...]
    x1 = jnp.maximum(h, 0.0)
    x2 = jnp.dot(x1, w2_ref[...],
                 preferred_element_type=jnp.float32) + b2_ref[...]
    mu = jnp.mean(x2, axis=-1, keepdims=True)
    xc = x2 - mu
    var = jnp.mean(xc * xc, axis=-1, keepdims=True)
    ne = xc * jax.lax.rsqrt(var + 1e-3) * g_ref[...] + be_ref[...]
    ne_ref[...] = ne
    newe_ref[...] = ne + e


def _edge_mlp(ga, gb, e, w1c, b1, w2, b2, g, be, blk_off=0, n_blk=None):
    num_e, d = e.shape
    be_blk = 2000
    if n_blk is None:
        n_blk = num_e // be_blk
    row = lambda i: (i + blk_off, 0)
    full = lambda i: (0, 0)
    ne, newe = pl.pallas_call(
        _edge_body,
        grid=(n_blk,),
        in_specs=[
            pl.BlockSpec((be_blk, d), row),
            pl.BlockSpec((be_blk, d), row),
            pl.BlockSpec((be_blk, d), row),
            pl.BlockSpec((d, d), full),
            pl.BlockSpec((1, d), full),
            pl.BlockSpec((d, d), full),
            pl.BlockSpec((1, d), full),
            pl.BlockSpec((1, d), full),
            pl.BlockSpec((1, d), full),
        ],
        out_specs=[
            pl.BlockSpec((be_blk, d), lambda i: (i, 0)),
            pl.BlockSpec((be_blk, d), lambda i: (i, 0)),
        ],
        out_shape=[
            jax.ShapeDtypeStruct((n_blk * be_blk, d), jnp.float32),
            jax.ShapeDtypeStruct((n_blk * be_blk, d), jnp.float32),
        ],
    )(ga, gb, e, w1c, b1, w2, b2, g, be)
    return ne, newe


def _node_body(x_ref, a0_ref, a1_ref, wa_ref, wb_ref, wc_ref, b1_ref,
               w2_ref, b2_ref, g_ref, be_ref, o_ref):
    x = x_ref[...]
    agg0 = a0_ref[0] + a0_ref[1]
    agg1 = a1_ref[0] + a1_ref[1]
    h = (jnp.dot(x, wa_ref[...], preferred_element_type=jnp.float32)
         + jnp.dot(agg0, wb_ref[...], preferred_element_type=jnp.float32)
         + jnp.dot(agg1, wc_ref[...], preferred_element_type=jnp.float32)
         + b1_ref[...])
    x1 = jnp.maximum(h, 0.0)
    x2 = jnp.dot(x1, w2_ref[...],
                 preferred_element_type=jnp.float32) + b2_ref[...]
    mu = jnp.mean(x2, axis=-1, keepdims=True)
    xc = x2 - mu
    var = jnp.mean(xc * xc, axis=-1, keepdims=True)
    nn = xc * jax.lax.rsqrt(var + 1e-3) * g_ref[...] + be_ref[...]
    o_ref[...] = nn + x


def _node_mlp(nodes, agg0p, agg1p, wa, wb, wc, b1, w2, b2, g, be):
    n, d = nodes.shape
    bn = 2000
    row = lambda i: (i, 0)
    prow = lambda i: (0, i, 0)
    full = lambda i: (0, 0)
    return pl.pallas_call(
        _node_body,
        grid=(n // bn,),
        in_specs=[
            pl.BlockSpec((bn, d), row),
            pl.BlockSpec((2, bn, d), prow),
            pl.BlockSpec((2, bn, d), prow),
            pl.BlockSpec((d, d), full),
            pl.BlockSpec((d, d), full),
            pl.BlockSpec((d, d), full),
            pl.BlockSpec((1, d), full),
            pl.BlockSpec((d, d), full),
            pl.BlockSpec((1, d), full),
            pl.BlockSpec((1, d), full),
            pl.BlockSpec((1, d), full),
        ],
        out_specs=pl.BlockSpec((bn, d), row),
        out_shape=jax.ShapeDtypeStruct((n, d), jnp.float32),
    )(nodes, agg0p, agg1p, wa, wb, wc, b1, w2, b2, g, be)


_GATHER_W = 160


def _sc_gather_pair(table_a, table_b, senders, receivers):
    num_e = senders.shape[0]
    d = table_a.shape[1]
    n_win = num_e // _GATHER_W
    idx_s = senders.reshape(n_win, 1, _GATHER_W)
    idx_r = receivers.reshape(n_win, 1, _GATHER_W)
    mesh = plsc.VectorSubcoreMesh(core_axis_name="core",
                                  subcore_axis_name="subcore")
    out_t = jax.ShapeDtypeStruct((num_e, d), table_a.dtype)

    @functools.partial(
        pl.kernel, out_type=[out_t, out_t], mesh=mesh,
        scratch_types=[pltpu.SemaphoreType.DMA, pltpu.SemaphoreType.DMA])
    def k(a_hbm, b_hbm, is_hbm, ir_hbm, oa_hbm, ob_hbm, sem_a, sem_b):
        def body(is_v, ir_v, oa_v, ob_v):
            da = pltpu.async_copy(a_hbm.at[is_v.at[0, 0]], oa_v, sem_a)
            db = pltpu.async_copy(b_hbm.at[ir_v.at[0, 0]], ob_v, sem_b)
            da.wait()
            db.wait()

        pltpu.emit_pipeline(
            body,
            grid=(n_win,),
            in_specs=[
                pl.BlockSpec((1, 1, _GATHER_W), lambda i: (i, 0, 0)),
                pl.BlockSpec((1, 1, _GATHER_W), lambda i: (i, 0, 0)),
            ],
            out_specs=[
                pl.BlockSpec((_GATHER_W, d), lambda i: (i, 0)),
                pl.BlockSpec((_GATHER_W, d), lambda i: (i, 0)),
            ],
            core_axis_name=("core", "subcore"),
            dimension_semantics=(pltpu.PARALLEL,),
        )(is_hbm, ir_hbm, oa_hbm, ob_hbm)

    return k(table_a, table_b, idx_s, idx_r)


_SCAT_W = 160


def _sc_segment_sum(ne, receivers, zeros_nd):
    num_e, d = ne.shape
    n = zeros_nd.shape[0]
    n_chunk = num_e // _SCAT_W
    idx3 = receivers.reshape(n_chunk, 1, _SCAT_W)
    mesh = plsc.VectorSubcoreMesh(core_axis_name="core",
                                  subcore_axis_name="subcore")

    @functools.partial(
        pl.kernel,
        out_type=jax.ShapeDtypeStruct((2, n, d), jnp.float32),
        mesh=mesh,
        scratch_types=[pltpu.VMEM_SHARED((n, d), jnp.float32)],
    )
    def k(ne_hbm, idx_hbm, z_hbm, out_hbm, acc_sh):
        cid = jax.lax.axis_index("core")
        sid = jax.lax.axis_index("subcore")

        @pl.when(sid == 0)
        def _():
            pltpu.sync_copy(z_hbm, acc_sh)

        plsc.subcore_barrier()

        def body(idx_v, rows_v):
            pltpu.sync_copy(rows_v, acc_sh.at[idx_v.at[0, 0]], add=True)

        pltpu.emit_pipeline(
            body,
            grid=(n_chunk,),
            in_specs=[
                pl.BlockSpec((1, 1, _SCAT_W), lambda i: (i, 0, 0)),
                pl.BlockSpec((_SCAT_W, d), lambda i: (i, 0)),
            ],
            out_specs=[],
            core_axis_name=("core", "subcore"),
            dimension_semantics=(pltpu.PARALLEL,),
        )(idx_hbm, ne_hbm)

        plsc.subcore_barrier()

        @pl.when(sid == 0)
        def _():
            pltpu.sync_copy(acc_sh, out_hbm.at[cid])

    return k(ne, idx3, zeros_nd)


def kernel(node_features, edge_feat_0, edge_feat_1, senders_0, receivers_0,
           senders_1, receivers_1, e0_W1, e0_b1, e0_W2, e0_b2, e0_g, e0_be,
           e1_W1, e1_b1, e1_W2, e1_b2, e1_g, e1_be, n_W1, n_b1, n_W2, n_b2,
           n_g, n_be):
    d = _D
    n = node_features.shape[0]


    w_cat = jnp.concatenate(
        [e0_W1[:d], e0_W1[d:2 * d], e1_W1[:d], e1_W1[d:2 * d]], axis=1)
    pa0, pb0, pa1, pb1 = _project_tables(node_features, w_cat)

    r2 = lambda v: v.reshape(1, d)

    ga0, gb0 = _sc_gather_pair(pa0, pb0, senders_0, receivers_0)
    ga1, gb1 = _sc_gather_pair(pa1, pb1, senders_1, receivers_1)

    ne0, new_e0 = _edge_mlp(ga0, gb0, edge_feat_0, e0_W1[2 * d:], r2(e0_b1),
                            e0_W2, r2(e0_b2), r2(e0_g), r2(e0_be))
    ne1, new_e1 = _edge_mlp(ga1, gb1, edge_feat_1, e1_W1[2 * d:], r2(e1_b1),
                            e1_W2, r2(e1_b2), r2(e1_g), r2(e1_be))

    zeros_nd = jnp.zeros((n, d), jnp.float32)
    agg0p = _sc_segment_sum(ne0, receivers_0, zeros_nd)
    agg1p = _sc_segment_sum(ne1, receivers_1, zeros_nd)

    new_nodes = _node_mlp(node_features, agg0p, agg1p, n_W1[:d],
                          n_W1[d:2 * d], n_W1[2 * d:], r2(n_b1), n_W2,
                          r2(n_b2), r2(n_g), r2(n_be))
    return (new_nodes, new_e0, new_e1)


# speedup vs baseline: 1.0102x; 1.0003x over previous
import functools

import jax
import jax.numpy as jnp
from jax.experimental import pallas as pl
from jax.experimental.pallas import tpu as pltpu
from jax.experimental.pallas import tpu_sc as plsc

_D = 128


def _proj_body(x_ref, w_ref, o0_ref, o1_ref, o2_ref, o3_ref):
    d = x_ref.shape[1]
    prod = jnp.dot(x_ref[...], w_ref[...], preferred_element_type=jnp.float32)
    o0_ref[...] = prod[:, :d]
    o1_ref[...] = prod[:, d:2 * d]
    o2_ref[...] = prod[:, 2 * d:3 * d]
    o3_ref[...] = prod[:, 3 * d:]


def _project_tables(nodes, w_cat):
    n, d = nodes.shape
    bn = 2000
    row = lambda i: (i, 0)
    out_t = jax.ShapeDtypeStruct((n, d), jnp.float32)
    return pl.pallas_call(
        _proj_body,
        grid=(n // bn,),
        in_specs=[
            pl.BlockSpec((bn, d), row),
            pl.BlockSpec((d, w_cat.shape[1]), lambda i: (0, 0)),
        ],
        out_specs=[pl.BlockSpec((bn, d), row)] * 4,
        out_shape=[out_t] * 4,
    )(nodes, w_cat)


def _edge_body(ga_ref, gb_ref, e_ref, w1c_ref, b1_ref, w2_ref, b2_ref,
               g_ref, be_ref, ne_ref, newe_ref):
    e = e_ref[...]
    gsum = ga_ref[...] + gb_ref[...]
    h = gsum + jnp.dot(
        e, w1c_ref[...], preferred_element_type=jnp.float32) + b1_ref[...]
    x1 = jnp.maximum(h, 0.0)
    x2 = jnp.dot(x1, w2_ref[...],
                 preferred_element_type=jnp.float32) + b2_ref[...]
    mu = jnp.mean(x2, axis=-1, keepdims=True)
    xc = x2 - mu
    var = jnp.mean(xc * xc, axis=-1, keepdims=True)
    ne = xc * jax.lax.rsqrt(var + 1e-3) * g_ref[...] + be_ref[...]
    ne_ref[...] = ne
    newe_ref[...] = ne + e


def _edge_mlp(ga, gb, e, w1c, b1, w2, b2, g, be, blk_off=0, n_blk=None):
    num_e, d = e.shape
    be_blk = 2000
    if n_blk is None:
        n_blk = num_e // be_blk
    row = lambda i: (i + blk_off, 0)
    full = lambda i: (0, 0)
    ne, newe = pl.pallas_call(
        _edge_body,
        grid=(n_blk,),
        in_specs=[
            pl.BlockSpec((be_blk, d), row),
            pl.BlockSpec((be_blk, d), row),
            pl.BlockSpec((be_blk, d), row),
            pl.BlockSpec((d, d), full),
            pl.BlockSpec((1, d), full),
            pl.BlockSpec((d, d), full),
            pl.BlockSpec((1, d), full),
            pl.BlockSpec((1, d), full),
            pl.BlockSpec((1, d), full),
        ],
        out_specs=[
            pl.BlockSpec((be_blk, d), lambda i: (i, 0)),
            pl.BlockSpec((be_blk, d), lambda i: (i, 0)),
        ],
        out_shape=[
            jax.ShapeDtypeStruct((n_blk * be_blk, d), jnp.float32),
            jax.ShapeDtypeStruct((n_blk * be_blk, d), jnp.float32),
        ],
    )(ga, gb, e, w1c, b1, w2, b2, g, be)
    return ne, newe


def _node_body(x_ref, a0_ref, a1_ref, wa_ref, wb_ref, wc_ref, b1_ref,
               w2_ref, b2_ref, g_ref, be_ref, o_ref):
    x = x_ref[...]
    agg0 = a0_ref[0] + a0_ref[1]
    agg1 = a1_ref[0] + a1_ref[1]
    h = (jnp.dot(x, wa_ref[...], preferred_element_type=jnp.float32)
         + jnp.dot(agg0, wb_ref[...], preferred_element_type=jnp.float32)
         + jnp.dot(agg1, wc_ref[...], preferred_element_type=jnp.float32)
         + b1_ref[...])
    x1 = jnp.maximum(h, 0.0)
    x2 = jnp.dot(x1, w2_ref[...],
                 preferred_element_type=jnp.float32) + b2_ref[...]
    mu = jnp.mean(x2, axis=-1, keepdims=True)
    xc = x2 - mu
    var = jnp.mean(xc * xc, axis=-1, keepdims=True)
    nn = xc * jax.lax.rsqrt(var + 1e-3) * g_ref[...] + be_ref[...]
    o_ref[...] = nn + x


def _node_mlp(nodes, agg0p, agg1p, wa, wb, wc, b1, w2, b2, g, be):
    n, d = nodes.shape
    bn = 2000
    row = lambda i: (i, 0)
    prow = lambda i: (0, i, 0)
    full = lambda i: (0, 0)
    return pl.pallas_call(
        _node_body,
        grid=(n // bn,),
        in_specs=[
            pl.BlockSpec((bn, d), row),
            pl.BlockSpec((2, bn, d), prow),
            pl.BlockSpec((2, bn, d), prow),
            pl.BlockSpec((d, d), full),
            pl.BlockSpec((d, d), full),
            pl.BlockSpec((d, d), full),
            pl.BlockSpec((1, d), full),
            pl.BlockSpec((d, d), full),
            pl.BlockSpec((1, d), full),
            pl.BlockSpec((1, d), full),
            pl.BlockSpec((1, d), full),
        ],
        out_specs=pl.BlockSpec((bn, d), row),
        out_shape=jax.ShapeDtypeStruct((n, d), jnp.float32),
    )(nodes, agg0p, agg1p, wa, wb, wc, b1, w2, b2, g, be)


def _sc_vector_warmup(table):
    d = table.shape[1]
    idx = jnp.zeros((1, 1, 8), jnp.int32)
    mesh = plsc.VectorSubcoreMesh(core_axis_name="core",
                                  subcore_axis_name="subcore")

    @functools.partial(
        pl.kernel, out_type=jax.ShapeDtypeStruct((8, d), table.dtype),
        mesh=mesh,
        scratch_types=[pltpu.VMEM((1, 8), jnp.int32),
                       pltpu.VMEM((8, d), jnp.float32),
                       pltpu.SemaphoreType.DMA])
    def k(t_hbm, i_hbm, o_hbm, idx_v, rows_v, sem):
        cid = jax.lax.axis_index("core")
        sid = jax.lax.axis_index("subcore")

        @pl.when((cid == 0) & (sid == 0))
        def _():
            pltpu.sync_copy(i_hbm.at[0], idx_v)
            pltpu.async_copy(t_hbm.at[idx_v.at[0]], rows_v, sem).wait()
            pltpu.sync_copy(rows_v, o_hbm)

    return k(table, idx)


_GATHER_W = 160


def _sc_gather_pair(table_a, table_b, senders, receivers):
    num_e = senders.shape[0]
    d = table_a.shape[1]
    n_win = num_e // _GATHER_W
    idx_s = senders.reshape(n_win, 1, _GATHER_W)
    idx_r = receivers.reshape(n_win, 1, _GATHER_W)
    mesh = plsc.VectorSubcoreMesh(core_axis_name="core",
                                  subcore_axis_name="subcore")
    out_t = jax.ShapeDtypeStruct((num_e, d), table_a.dtype)

    @functools.partial(
        pl.kernel, out_type=[out_t, out_t], mesh=mesh,
        scratch_types=[pltpu.SemaphoreType.DMA, pltpu.SemaphoreType.DMA])
    def k(a_hbm, b_hbm, is_hbm, ir_hbm, oa_hbm, ob_hbm, sem_a, sem_b):
        def body(is_v, ir_v, oa_v, ob_v):
            da = pltpu.async_copy(a_hbm.at[is_v.at[0, 0]], oa_v, sem_a)
            db = pltpu.async_copy(b_hbm.at[ir_v.at[0, 0]], ob_v, sem_b)
            da.wait()
            db.wait()

        pltpu.emit_pipeline(
            body,
            grid=(n_win,),
            in_specs=[
                pl.BlockSpec((1, 1, _GATHER_W), lambda i: (i, 0, 0)),
                pl.BlockSpec((1, 1, _GATHER_W), lambda i: (i, 0, 0)),
            ],
            out_specs=[
                pl.BlockSpec((_GATHER_W, d), lambda i: (i, 0)),
                pl.BlockSpec((_GATHER_W, d), lambda i: (i, 0)),
            ],
            core_axis_name=("core", "subcore"),
            dimension_semantics=(pltpu.PARALLEL,),
        )(is_hbm, ir_hbm, oa_hbm, ob_hbm)

    return k(table_a, table_b, idx_s, idx_r)


_SCAT_W = 160


def _sc_segment_sum(ne, receivers, zeros_nd):
    num_e, d = ne.shape
    n = zeros_nd.shape[0]
    n_chunk = num_e // _SCAT_W
    idx3 = receivers.reshape(n_chunk, 1, _SCAT_W)
    mesh = plsc.VectorSubcoreMesh(core_axis_name="core",
                                  subcore_axis_name="subcore")

    @functools.partial(
        pl.kernel,
        out_type=jax.ShapeDtypeStruct((2, n, d), jnp.float32),
        mesh=mesh,
        scratch_types=[pltpu.VMEM_SHARED((n, d), jnp.float32)],
    )
    def k(ne_hbm, idx_hbm, z_hbm, out_hbm, acc_sh):
        cid = jax.lax.axis_index("core")
        sid = jax.lax.axis_index("subcore")

        @pl.when(sid == 0)
        def _():
            pltpu.sync_copy(z_hbm, acc_sh)

        plsc.subcore_barrier()

        def body(idx_v, rows_v):
            pltpu.sync_copy(rows_v, acc_sh.at[idx_v.at[0, 0]], add=True)

        pltpu.emit_pipeline(
            body,
            grid=(n_chunk,),
            in_specs=[
                pl.BlockSpec((1, 1, _SCAT_W), lambda i: (i, 0, 0)),
                pl.BlockSpec((_SCAT_W, d), lambda i: (i, 0)),
            ],
            out_specs=[],
            core_axis_name=("core", "subcore"),
            dimension_semantics=(pltpu.PARALLEL,),
        )(idx_hbm, ne_hbm)

        plsc.subcore_barrier()

        @pl.when(sid == 0)
        def _():
            pltpu.sync_copy(acc_sh, out_hbm.at[cid])

    return k(ne, idx3, zeros_nd)


def kernel(node_features, edge_feat_0, edge_feat_1, senders_0, receivers_0,
           senders_1, receivers_1, e0_W1, e0_b1, e0_W2, e0_b2, e0_g, e0_be,
           e1_W1, e1_b1, e1_W2, e1_b2, e1_g, e1_be, n_W1, n_b1, n_W2, n_b2,
           n_g, n_be):
    d = _D
    n = node_features.shape[0]


    w_cat = jnp.concatenate(
        [e0_W1[:d], e0_W1[d:2 * d], e1_W1[:d], e1_W1[d:2 * d]], axis=1)
    pa0, pb0, pa1, pb1 = _project_tables(node_features, w_cat)

    r2 = lambda v: v.reshape(1, d)

    ga0, gb0 = _sc_gather_pair(pa0, pb0, senders_0, receivers_0)
    ga1, gb1 = _sc_gather_pair(pa1, pb1, senders_1, receivers_1)

    ne0, new_e0 = _edge_mlp(ga0, gb0, edge_feat_0, e0_W1[2 * d:], r2(e0_b1),
                            e0_W2, r2(e0_b2), r2(e0_g), r2(e0_be))
    ne1, new_e1 = _edge_mlp(ga1, gb1, edge_feat_1, e1_W1[2 * d:], r2(e1_b1),
                            e1_W2, r2(e1_b2), r2(e1_g), r2(e1_be))

    zeros_nd = jnp.zeros((n, d), jnp.float32)
    agg0p = _sc_segment_sum(ne0, receivers_0, zeros_nd)
    agg1p = _sc_segment_sum(ne1, receivers_1, zeros_nd)

    new_nodes = _node_mlp(node_features, agg0p, agg1p, n_W1[:d],
                          n_W1[d:2 * d], n_W1[2 * d:], r2(n_b1), n_W2,
                          r2(n_b2), r2(n_g), r2(n_be))
    return (new_nodes, new_e0, new_e1)
